# baseline (speedup 1.0000x reference)
import dataclasses
import functools

import jax
import jax.numpy as jnp
from jax import lax
from jax.experimental import pallas as pl
from jax.experimental.pallas import tpu as pltpu
from jax.experimental.pallas import tpu_sc as plsc

_CHUNK = 128
_BE = 2000
_BN = 2000


def _matmul(x, w):
    n, d = x.shape

    def body(x_ref, w_ref, o_ref):
        o_ref[...] = jnp.dot(x_ref[...], w_ref[...],
                             preferred_element_type=jnp.float32)

    return pl.pallas_call(
        body,
        grid=(n // _BN,),
        in_specs=[
            pl.BlockSpec((_BN, d), lambda i: (i, 0)),
            pl.BlockSpec((d, d), lambda i: (0, 0)),
        ],
        out_specs=pl.BlockSpec((_BN, d), lambda i: (i, 0)),
        out_shape=jax.ShapeDtypeStruct((n, d), jnp.float32),
    )(x, w)


def _sc_gather(table_h, table_r, hidx2, ridx2, cidx2):
    nchunk, _ = hidx2.shape
    d = table_h.shape[1]
    info = plsc.get_sparse_core_info()
    ncg = 1
    nw = ncg * info.num_subcores
    cpw = nchunk // nw
    e = nchunk * _CHUNK
    mesh = plsc.VectorSubcoreMesh(core_axis_name="c", subcore_axis_name="s",
                                  num_cores=ncg)
    out_t = jax.ShapeDtypeStruct((e, d), jnp.float32)

    @functools.partial(
        pl.kernel, mesh=mesh,
        out_type=[out_t, out_t, out_t],
        scratch_types=[
            pltpu.VMEM((cpw, _CHUNK), jnp.int32),
            pltpu.VMEM((cpw, _CHUNK), jnp.int32),
            pltpu.VMEM((cpw, _CHUNK), jnp.int32),
        ] + [pltpu.VMEM((_CHUNK, d), jnp.float32)] * 6
          + [pltpu.SemaphoreType.DMA] * 6,
    )
    def k(th_hbm, tr_hbm, hi_hbm, ri_hbm, ci_hbm, ho_hbm, ro_hbm, to_hbm,
          hib, rib, cib, b0, b1, b2, b3, b4, b5, s0, s1, s2, s3, s4, s5):
        w = lax.axis_index("s") * ncg + lax.axis_index("c")
        c0 = w * cpw
        pltpu.sync_copy(hi_hbm.at[pl.ds(c0, cpw)], hib)
        pltpu.sync_copy(ri_hbm.at[pl.ds(c0, cpw)], rib)
        pltpu.sync_copy(ci_hbm.at[pl.ds(c0, cpw)], cib)
        bufs = (b0, b1, b2, b3, b4, b5)
        sems = (s0, s1, s2, s3, s4, s5)
        tabs = (th_hbm, tr_hbm, th_hbm)
        idxs = (hib, rib, cib)
        outs = (ho_hbm, ro_hbm, to_hbm)
        for jo in range(2):
            for src in range(3):
                bi = jo * 3 + src
                pltpu.async_copy(tabs[src].at[idxs[src].at[jo]],
                                 bufs[bi], sems[bi])

        @pl.loop(0, cpw, step=2)
        def _(jj):
            for jo in range(2):
                j = jj + jo
                gbase = (c0 + j) * _CHUNK
                for src in range(3):
                    bi = jo * 3 + src
                    pltpu.make_async_copy(tabs[src].at[idxs[src].at[j]],
                                          bufs[bi], sems[bi]).wait()
                    pltpu.async_copy(bufs[bi],
                                     outs[src].at[pl.ds(gbase, _CHUNK)],
                                     sems[bi])
            for jo in range(2):
                jn = jj + jo + 2

                @pl.when(jn < cpw)
                def _():
                    for src in range(3):
                        bi = jo * 3 + src
                        gprev = (c0 + jn - 2) * _CHUNK
                        pltpu.make_async_copy(
                            bufs[bi], outs[src].at[pl.ds(gprev, _CHUNK)],
                            sems[bi]).wait()
                        pltpu.async_copy(tabs[src].at[idxs[src].at[jn]],
                                         bufs[bi], sems[bi])

        for jo in range(2):
            for src in range(3):
                bi = jo * 3 + src
                pltpu.make_async_copy(bufs[bi],
                                      outs[src].at[pl.ds(0, _CHUNK)],
                                      sems[bi]).wait()

    return k(table_h, table_r, hidx2, ridx2, cidx2)


def _conv_channels(hs, rs, ts, cw_ref):
    outs = []
    for c in range(4):
        acc = None
        for ki in range(3):
            term = (cw_ref[c * 9 + ki * 3 + 0] * hs[ki]
                    + cw_ref[c * 9 + ki * 3 + 1] * rs[ki]
                    + cw_ref[c * 9 + ki * 3 + 2] * ts[ki])
            acc = term if acc is None else acc + term
        outs.append(acc)
    return outs


def _stats_pass(H, R, T, cwf, ne):
    d = H.shape[1]
    nb = ne // _BE

    def body(cw_ref, h_ref, r_ref, t_ref, o_ref):
        i = pl.program_id(0)

        @pl.when(i == 0)
        def _():
            o_ref[...] = jnp.zeros_like(o_ref)

        h = h_ref[...].astype(jnp.float32)
        r = r_ref[...].astype(jnp.float32)
        t = t_ref[...].astype(jnp.float32)
        vals = [
            jnp.sum(h) + jnp.sum(r) + jnp.sum(t),
            jnp.sum(h * h) + jnp.sum(r * r) + jnp.sum(t * t),
        ]
        hs = [h[:, ki:ki + d - 2] for ki in range(3)]
        rs = [r[:, ki:ki + d - 2] for ki in range(3)]
        ts = [t[:, ki:ki + d - 2] for ki in range(3)]
        for L in _conv_channels(hs, rs, ts, cw_ref):
            vals.append(jnp.sum(L))
            vals.append(jnp.sum(L * L))
        lane = lax.broadcasted_iota(jnp.int32, (1, 128), 1)
        p = jnp.zeros((1, 128), jnp.float32)
        for k, v in enumerate(vals):
            p = p + jnp.where(lane == k, v, 0.0)
        o_ref[...] += p

    return pl.pallas_call(
        body,
        grid=(nb,),
        in_specs=[
            pl.BlockSpec(memory_space=pltpu.SMEM),
            pl.BlockSpec((_BE, d), lambda i: (i, 0)),
            pl.BlockSpec((_BE, d), lambda i: (i, 0)),
            pl.BlockSpec((_BE, d), lambda i: (i, 0)),
        ],
        out_specs=pl.BlockSpec((1, 128), lambda i: (0, 0)),
        out_shape=jax.ShapeDtypeStruct((1, 128), jnp.float32),
    )(cwf, H, R, T)


def _score_pass(H, R, T, cwf, scal, ne):
    d = H.shape[1]
    nb = ne // _BE
    dc = d - 2

    def body(cw_ref, h_ref, r_ref, t_ref, scal_ref, o_ref):
        h = h_ref[...].astype(jnp.float32)
        r = r_ref[...].astype(jnp.float32)
        t = t_ref[...].astype(jnp.float32)
        hs = [h[:, ki:ki + dc] for ki in range(3)]
        rs = [r[:, ki:ki + dc] for ki in range(3)]
        ts = [t[:, ki:ki + dc] for ki in range(3)]
        tot = jnp.zeros((h.shape[0], dc), jnp.float32)
        for c, L in enumerate(_conv_channels(hs, rs, ts, cw_ref)):
            z = L * scal_ref[c:c + 1, :dc] + scal_ref[4 + c:5 + c, :dc]
            tot = tot + jnp.maximum(z, 0.0) * scal_ref[8 + c:9 + c, :dc]
        ev = jnp.sum(tot, axis=1, keepdims=True)
        ev = jnp.where(ev >= 0.0, ev, 0.01 * ev)
        o_ref[...] = jnp.exp(ev)

    return pl.pallas_call(
        body,
        grid=(nb,),
        in_specs=[
            pl.BlockSpec(memory_space=pltpu.SMEM),
            pl.BlockSpec((_BE, d), lambda i: (i, 0)),
            pl.BlockSpec((_BE, d), lambda i: (i, 0)),
            pl.BlockSpec((_BE, d), lambda i: (i, 0)),
            pl.BlockSpec((16, 128), lambda i: (0, 0)),
        ],
        out_specs=pl.BlockSpec((_BE, 1), lambda i: (i, 0)),
        out_shape=jax.ShapeDtypeStruct((ne, 1), jnp.float32),
    )(cwf, H, R, T, scal)


def _sc_aggregate(ex2, row2, col2, input_):
    nchunk, _ = ex2.shape
    ck = 64
    n, d = input_.shape
    info = plsc.get_sparse_core_info()
    ncores, nsub = info.num_cores, info.num_subcores
    nw = ncores * nsub
    cpw = nchunk // nw
    nck = cpw * 2
    rcp = 40
    nrchunk = n // rcp
    iters_z = (nrchunk + nsub - 1) // nsub
    ndr = -(-n // d) + (-(-n // d)) % 8
    mesh = plsc.VectorSubcoreMesh(core_axis_name="c", subcore_axis_name="s")
    cp = pltpu.CompilerParams()
    if "needs_layout_passes" in pltpu.CompilerParams.__dataclass_fields__:
        cp = dataclasses.replace(cp, needs_layout_passes=False)

    @functools.partial(
        pl.kernel, mesh=mesh, compiler_params=cp,
        out_type=[jax.ShapeDtypeStruct((ncores, n, d), jnp.float32),
                  jax.ShapeDtypeStruct((ncores, ndr, d), jnp.float32)],
        scratch_types=[
            pltpu.VMEM((cpw, 128), jnp.float32),
            pltpu.VMEM((cpw, 128), jnp.int32),
            pltpu.VMEM((cpw, 128), jnp.int32),
            pltpu.VMEM((ck,), jnp.float32),
            pltpu.VMEM((ck,), jnp.float32),
            pltpu.VMEM((ck,), jnp.int32),
            pltpu.VMEM((ck,), jnp.int32),
            pltpu.VMEM((ck,), jnp.int32),
            pltpu.VMEM((ck,), jnp.int32),
        ] + [pltpu.VMEM((ck, d), jnp.float32)] * 4 + [
            pltpu.VMEM_SHARED((n, d), jnp.float32),
            pltpu.VMEM_SHARED((ndr, d), jnp.float32),
            pltpu.SemaphoreType.DMA,
            pltpu.SemaphoreType.DMA,
            pltpu.SemaphoreType.DMA,
        ],
    )
    def k(ex_hbm, row_hbm, col_hbm, in_hbm, agg_hbm, den_hbm,
          exb, rowb, colb, ec0, ec1, rc0, rc1, rd0, rd1, r0, r1, vv, u0,
          agg_sh, den_sh, gs0, gs1, ssem):
        cid = lax.axis_index("c")
        sid = lax.axis_index("s")
        w = sid * ncores + cid
        c0 = w * cpw
        pltpu.sync_copy(ex_hbm.at[pl.ds(c0, cpw)], exb)
        pltpu.sync_copy(row_hbm.at[pl.ds(c0, cpw)], rowb)
        pltpu.sync_copy(col_hbm.at[pl.ds(c0, cpw)], colb)

        @pl.loop(0, ck)
        def _(i):
            for kk in range(d // 16):
                u0[i, pl.ds(kk * 16, 16)] = jnp.zeros((16,), jnp.float32)

        @pl.loop(0, iters_z)
        def _(z):
            c = sid + z * nsub

            @pl.when(c < nrchunk)
            def _():
                pltpu.sync_copy(u0.at[pl.ds(0, rcp)],
                                agg_sh.at[pl.ds(c * rcp, rcp)])

        @pl.when(sid == 0)
        def _():
            pltpu.sync_copy(u0.at[pl.ds(0, ndr - ck)],
                            den_sh.at[pl.ds(0, ndr - ck)])
            pltpu.sync_copy(u0.at[pl.ds(0, ck)], den_sh.at[pl.ds(ndr - ck, ck)])

        plsc.subcore_barrier()

        rows = (r0, r1)
        excs = (ec0, ec1)
        rowcs = (rc0, rc1)
        rowds = (rd0, rd1)
        gsems = (gs0, gs1)
        for jo in range(2):
            pltpu.async_copy(in_hbm.at[colb.at[0, pl.ds(jo * ck, ck)]],
                             rows[jo], gsems[jo])
        iota16 = lax.broadcasted_iota(jnp.int32, (16,), 0)

        @pl.loop(0, nck, step=2)
        def _(jj):
            for jo in range(2):
                j = jj + jo
                jh = lax.shift_right_logical(j, 1)
                off = lax.rem(j, 2) * ck
                pltpu.make_async_copy(
                    in_hbm.at[colb.at[jh, pl.ds(off, ck)]],
                    rows[jo], gsems[jo]).wait()

                for kk in range(ck // 16):
                    r16 = rowb[jh, pl.ds(off + kk * 16, 16)]
                    rowcs[jo][pl.ds(kk * 16, 16)] = r16
                    rowds[jo][pl.ds(kk * 16, 16)] = (
                        lax.shift_right_logical(r16, 7))
                    excs[jo][pl.ds(kk * 16, 16)] = (
                        exb[jh, pl.ds(off + kk * 16, 16)])

                @pl.when(j > 0)
                def _():
                    pltpu.make_async_copy(vv, agg_sh.at[pl.ds(0, ck)],
                                          ssem).wait()
                    pltpu.make_async_copy(u0, agg_sh.at[pl.ds(0, ck)],
                                          ssem).wait()

                @pl.loop(0, ck)
                def _(i):
                    splat_i = jnp.full((16,), i, jnp.int32)
                    sv = plsc.load_gather(excs[jo], [splat_i])
                    rv = plsc.load_gather(rowcs[jo], [splat_i])
                    pv = lax.rem(rv, 128)
                    for kk in range(d // 16):
                        vv[i, pl.ds(kk * 16, 16)] = (
                            rows[jo][i, pl.ds(kk * 16, 16)] * sv)
                        u0[i, pl.ds(kk * 16, 16)] = jnp.where(
                            iota16 + (kk * 16) == pv, sv,
                            jnp.zeros((16,), jnp.float32))

                pltpu.async_copy(vv, agg_sh.at[rowcs[jo]], ssem, add=True)
                pltpu.async_copy(u0, den_sh.at[rowds[jo]], ssem, add=True)

                @pl.when(j + 2 < nck)
                def _():
                    jn = j + 2
                    jhn = lax.shift_right_logical(jn, 1)
                    offn = lax.rem(jn, 2) * ck
                    pltpu.async_copy(
                        in_hbm.at[colb.at[jhn, pl.ds(offn, ck)]],
                        rows[jo], gsems[jo])

        pltpu.make_async_copy(vv, agg_sh.at[pl.ds(0, ck)], ssem).wait()
        pltpu.make_async_copy(u0, agg_sh.at[pl.ds(0, ck)], ssem).wait()

        plsc.subcore_barrier()

        @pl.loop(0, iters_z)
        def _(z):
            c = sid + z * nsub

            @pl.when(c < nrchunk)
            def _():
                pltpu.sync_copy(agg_sh.at[pl.ds(c * rcp, rcp)],
                                agg_hbm.at[cid, pl.ds(c * rcp, rcp)])

        @pl.when(sid == 0)
        def _():
            pltpu.sync_copy(den_sh, den_hbm.at[cid])

    return k(ex2, row2, col2, input_)


def _finalize(input_, agg_pair, den_pair):
    n, d = input_.shape

    def body(x_ref, a_ref, dn_ref, o_ref):
        a = a_ref[0] + a_ref[1]
        den = dn_ref[0] + dn_ref[1]
        agg = jnp.where(den > 0.0, a / den, 0.0)
        out = x_ref[...] + agg
        o_ref[...] = jnp.where(out > 0.0, out, jnp.exp(out) - 1.0)

    return pl.pallas_call(
        body,
        grid=(n // _BN,),
        in_specs=[
            pl.BlockSpec((_BN, d), lambda i: (i, 0)),
            pl.BlockSpec((2, _BN, d), lambda i: (0, i, 0)),
            pl.BlockSpec((2, _BN, 1), lambda i: (0, i, 0)),
        ],
        out_specs=pl.BlockSpec((_BN, d), lambda i: (i, 0)),
        out_shape=jax.ShapeDtypeStruct((n, d), jnp.float32),
    )(input_, agg_pair, den_pair)


def kernel(input, triple, rel_table, W, conv_w, conv_b, bn1_gamma, bn1_beta,
           bn2_gamma, bn2_beta, fc_w):
    n, d = input.shape
    e = triple.shape[0]
    dc = d - 2
    eps = 1e-5

    row = triple[:, 0]
    rel = triple[:, 1]
    col = triple[:, 2]

    info = plsc.get_sparse_core_info()
    nw = info.num_cores * info.num_subcores
    per_w = nw * _CHUNK
    epad = -(-e // per_w) * per_w
    cpw = epad // per_w
    if cpw % 2:
        cpw += 1
        epad = cpw * per_w
    padlen = epad - e
    zpad = jnp.zeros((padlen,), jnp.int32)
    row2 = jnp.concatenate([row, zpad]).reshape(-1, _CHUNK)
    rel2 = jnp.concatenate([rel, zpad]).reshape(-1, _CHUNK)
    col2 = jnp.concatenate([col, zpad]).reshape(-1, _CHUNK)

    input_ = _matmul(input, W)

    Hb, Rb, Tb = _sc_gather(input_, rel_table, row2, rel2, col2)

    cwf = conv_w.reshape(4 * 9)
    stats = _stats_pass(Hb, Rb, Tb, cwf, e)[0]

    cnt1 = 3.0 * e * d
    mu1 = stats[0] / cnt1
    v1 = stats[1] / cnt1 - mu1 * mu1
    a1 = bn1_gamma[0] / jnp.sqrt(v1 + eps)
    cnt2 = float(e * dc)
    m0 = stats[2:10:2] / cnt2
    v0 = stats[3:10:2] / cnt2 - m0 * m0
    s_c = a1 * bn2_gamma / jnp.sqrt(a1 * a1 * v0 + eps)
    o_c = bn2_beta - s_c * m0

    fc2 = fc_w.reshape(4, dc)
    scal = jnp.zeros((16, 128), jnp.float32)
    scal = scal.at[0:4, :dc].set(jnp.broadcast_to(s_c[:, None], (4, dc)))
    scal = scal.at[4:8, :dc].set(jnp.broadcast_to(o_c[:, None], (4, dc)))
    scal = scal.at[8:12, :dc].set(fc2)

    ex = _score_pass(Hb, Rb, Tb, cwf, scal, e)[:, 0]
    ex2 = jnp.concatenate(
        [ex, jnp.zeros((padlen,), jnp.float32)]).reshape(-1, _CHUNK)

    agg_pair, den_out = _sc_aggregate(ex2, row2, col2, input_)
    den_pair = den_out.reshape(2, -1)[:, :n].reshape(2, n, 1)

    return _finalize(input_, agg_pair, den_pair)


# speedup vs baseline: 1.0200x; 1.0200x over previous
import dataclasses
import functools

import jax
import jax.numpy as jnp
from jax import lax
from jax.experimental import pallas as pl
from jax.experimental.pallas import tpu as pltpu
from jax.experimental.pallas import tpu_sc as plsc

_CHUNK = 128
_BE = 2000
_BN = 2000


def _matmul(x, w):
    n, d = x.shape

    def body(x_ref, w_ref, o_ref):
        o_ref[...] = jnp.dot(x_ref[...], w_ref[...],
                             preferred_element_type=jnp.float32)

    return pl.pallas_call(
        body,
        grid=(n // _BN,),
        in_specs=[
            pl.BlockSpec((_BN, d), lambda i: (i, 0)),
            pl.BlockSpec((d, d), lambda i: (0, 0)),
        ],
        out_specs=pl.BlockSpec((_BN, d), lambda i: (i, 0)),
        out_shape=jax.ShapeDtypeStruct((n, d), jnp.float32),
    )(x, w)


def _sc_gather(table_h, table_r, hidx2, ridx2, cidx2):
    nchunk, _ = hidx2.shape
    d = table_h.shape[1]
    info = plsc.get_sparse_core_info()
    nw = info.num_cores * info.num_subcores
    iters = nchunk // nw
    e = nchunk * _CHUNK
    mesh = plsc.VectorSubcoreMesh(core_axis_name="c", subcore_axis_name="s")
    out_t = jax.ShapeDtypeStruct((e, d), jnp.float32)

    @functools.partial(
        pl.kernel, mesh=mesh,
        out_type=[out_t, out_t, out_t],
        scratch_types=[
            pltpu.VMEM((_CHUNK,), jnp.int32),
            pltpu.VMEM((_CHUNK,), jnp.int32),
            pltpu.VMEM((_CHUNK,), jnp.int32),
            pltpu.VMEM((_CHUNK, d), jnp.float32),
            pltpu.VMEM((_CHUNK, d), jnp.float32),
            pltpu.VMEM((_CHUNK, d), jnp.float32),
            pltpu.SemaphoreType.DMA,
            pltpu.SemaphoreType.DMA,
            pltpu.SemaphoreType.DMA,
        ],
    )
    def k(th_hbm, tr_hbm, hi_hbm, ri_hbm, ci_hbm, ho_hbm, ro_hbm, to_hbm,
          hi_v, ri_v, ci_v, hb, rb, tb, s0, s1, s2):
        w = lax.axis_index("s") * info.num_cores + lax.axis_index("c")

        @pl.loop(0, iters)
        def _(jj):
            j = w + jj * nw
            base = j * _CHUNK
            pltpu.sync_copy(hi_hbm.at[j], hi_v)
            pltpu.sync_copy(ri_hbm.at[j], ri_v)
            pltpu.sync_copy(ci_hbm.at[j], ci_v)
            c0 = pltpu.async_copy(th_hbm.at[hi_v], hb, s0)
            c1 = pltpu.async_copy(tr_hbm.at[ri_v], rb, s1)
            c2 = pltpu.async_copy(th_hbm.at[ci_v], tb, s2)
            c0.wait()
            c1.wait()
            c2.wait()
            pltpu.sync_copy(hb, ho_hbm.at[pl.ds(base, _CHUNK)])
            pltpu.sync_copy(rb, ro_hbm.at[pl.ds(base, _CHUNK)])
            pltpu.sync_copy(tb, to_hbm.at[pl.ds(base, _CHUNK)])

    return k(table_h, table_r, hidx2, ridx2, cidx2)


def _conv_channels(hs, rs, ts, cw_ref):
    outs = []
    for c in range(4):
        acc = None
        for ki in range(3):
            term = (cw_ref[c * 9 + ki * 3 + 0] * hs[ki]
                    + cw_ref[c * 9 + ki * 3 + 1] * rs[ki]
                    + cw_ref[c * 9 + ki * 3 + 2] * ts[ki])
            acc = term if acc is None else acc + term
        outs.append(acc)
    return outs


def _stats_pass(H, R, T, cwf, ne):
    d = H.shape[1]
    nb = ne // _BE

    def body(cw_ref, h_ref, r_ref, t_ref, o_ref):
        i = pl.program_id(0)

        @pl.when(i == 0)
        def _():
            o_ref[...] = jnp.zeros_like(o_ref)

        h = h_ref[...].astype(jnp.float32)
        r = r_ref[...].astype(jnp.float32)
        t = t_ref[...].astype(jnp.float32)
        vals = [
            jnp.sum(h) + jnp.sum(r) + jnp.sum(t),
            jnp.sum(h * h) + jnp.sum(r * r) + jnp.sum(t * t),
        ]
        hs = [h[:, ki:ki + d - 2] for ki in range(3)]
        rs = [r[:, ki:ki + d - 2] for ki in range(3)]
        ts = [t[:, ki:ki + d - 2] for ki in range(3)]
        for L in _conv_channels(hs, rs, ts, cw_ref):
            vals.append(jnp.sum(L))
            vals.append(jnp.sum(L * L))
        lane = lax.broadcasted_iota(jnp.int32, (1, 128), 1)
        p = jnp.zeros((1, 128), jnp.float32)
        for k, v in enumerate(vals):
            p = p + jnp.where(lane == k, v, 0.0)
        o_ref[...] += p

    return pl.pallas_call(
        body,
        grid=(nb,),
        in_specs=[
            pl.BlockSpec(memory_space=pltpu.SMEM),
            pl.BlockSpec((_BE, d), lambda i: (i, 0)),
            pl.BlockSpec((_BE, d), lambda i: (i, 0)),
            pl.BlockSpec((_BE, d), lambda i: (i, 0)),
        ],
        out_specs=pl.BlockSpec((1, 128), lambda i: (0, 0)),
        out_shape=jax.ShapeDtypeStruct((1, 128), jnp.float32),
    )(cwf, H, R, T)


def _score_pass(H, R, T, cwf, scal, ne):
    d = H.shape[1]
    nb = ne // _BE
    dc = d - 2

    def body(cw_ref, h_ref, r_ref, t_ref, scal_ref, o_ref):
        h = h_ref[...].astype(jnp.float32)
        r = r_ref[...].astype(jnp.float32)
        t = t_ref[...].astype(jnp.float32)
        hs = [h[:, ki:ki + dc] for ki in range(3)]
        rs = [r[:, ki:ki + dc] for ki in range(3)]
        ts = [t[:, ki:ki + dc] for ki in range(3)]
        tot = jnp.zeros((h.shape[0], dc), jnp.float32)
        for c, L in enumerate(_conv_channels(hs, rs, ts, cw_ref)):
            z = L * scal_ref[c:c + 1, :dc] + scal_ref[4 + c:5 + c, :dc]
            tot = tot + jnp.maximum(z, 0.0) * scal_ref[8 + c:9 + c, :dc]
        ev = jnp.sum(tot, axis=1, keepdims=True)
        ev = jnp.where(ev >= 0.0, ev, 0.01 * ev)
        o_ref[...] = jnp.exp(ev)

    return pl.pallas_call(
        body,
        grid=(nb,),
        in_specs=[
            pl.BlockSpec(memory_space=pltpu.SMEM),
            pl.BlockSpec((_BE, d), lambda i: (i, 0)),
            pl.BlockSpec((_BE, d), lambda i: (i, 0)),
            pl.BlockSpec((_BE, d), lambda i: (i, 0)),
            pl.BlockSpec((16, 128), lambda i: (0, 0)),
        ],
        out_specs=pl.BlockSpec((_BE, 1), lambda i: (i, 0)),
        out_shape=jax.ShapeDtypeStruct((ne, 1), jnp.float32),
    )(cwf, H, R, T, scal)


def _sc_aggregate(ex2, row2, col2, input_):
    nchunk, _ = ex2.shape
    ck = 64
    n, d = input_.shape
    info = plsc.get_sparse_core_info()
    ncores, nsub = info.num_cores, info.num_subcores
    nw = ncores * nsub
    cpw = nchunk // nw
    nck = cpw * 2
    rcp = 40
    nrchunk = n // rcp
    iters_z = (nrchunk + nsub - 1) // nsub
    ndr = -(-n // d) + (-(-n // d)) % 8
    mesh = plsc.VectorSubcoreMesh(core_axis_name="c", subcore_axis_name="s")
    cp = pltpu.CompilerParams()
    if "needs_layout_passes" in pltpu.CompilerParams.__dataclass_fields__:
        cp = dataclasses.replace(cp, needs_layout_passes=False)

    @functools.partial(
        pl.kernel, mesh=mesh, compiler_params=cp,
        out_type=[jax.ShapeDtypeStruct((ncores, n, d), jnp.float32),
                  jax.ShapeDtypeStruct((ncores, ndr, d), jnp.float32)],
        scratch_types=[
            pltpu.VMEM((cpw, 128), jnp.float32),
            pltpu.VMEM((cpw, 128), jnp.int32),
            pltpu.VMEM((cpw, 128), jnp.int32),
            pltpu.VMEM((ck,), jnp.float32),
            pltpu.VMEM((ck,), jnp.float32),
            pltpu.VMEM((ck,), jnp.int32),
            pltpu.VMEM((ck,), jnp.int32),
            pltpu.VMEM((ck,), jnp.int32),
            pltpu.VMEM((ck,), jnp.int32),
        ] + [pltpu.VMEM((ck, d), jnp.float32)] * 4 + [
            pltpu.VMEM_SHARED((n, d), jnp.float32),
            pltpu.VMEM_SHARED((ndr, d), jnp.float32),
            pltpu.SemaphoreType.DMA,
            pltpu.SemaphoreType.DMA,
            pltpu.SemaphoreType.DMA,
        ],
    )
    def k(ex_hbm, row_hbm, col_hbm, in_hbm, agg_hbm, den_hbm,
          exb, rowb, colb, ec0, ec1, rc0, rc1, rd0, rd1, r0, r1, vv, u0,
          agg_sh, den_sh, gs0, gs1, ssem):
        cid = lax.axis_index("c")
        sid = lax.axis_index("s")
        w = sid * ncores + cid
        c0 = w * cpw
        pltpu.sync_copy(ex_hbm.at[pl.ds(c0, cpw)], exb)
        pltpu.sync_copy(row_hbm.at[pl.ds(c0, cpw)], rowb)
        pltpu.sync_copy(col_hbm.at[pl.ds(c0, cpw)], colb)

        @pl.loop(0, ck)
        def _(i):
            for kk in range(d // 16):
                u0[i, pl.ds(kk * 16, 16)] = jnp.zeros((16,), jnp.float32)

        @pl.loop(0, iters_z)
        def _(z):
            c = sid + z * nsub

            @pl.when(c < nrchunk)
            def _():
                pltpu.sync_copy(u0.at[pl.ds(0, rcp)],
                                agg_sh.at[pl.ds(c * rcp, rcp)])

        @pl.when(sid == 0)
        def _():
            pltpu.sync_copy(u0.at[pl.ds(0, ndr - ck)],
                            den_sh.at[pl.ds(0, ndr - ck)])
            pltpu.sync_copy(u0.at[pl.ds(0, ck)], den_sh.at[pl.ds(ndr - ck, ck)])

        plsc.subcore_barrier()

        rows = (r0, r1)
        excs = (ec0, ec1)
        rowcs = (rc0, rc1)
        rowds = (rd0, rd1)
        gsems = (gs0, gs1)
        for jo in range(2):
            pltpu.async_copy(in_hbm.at[colb.at[0, pl.ds(jo * ck, ck)]],
                             rows[jo], gsems[jo])
        iota16 = lax.broadcasted_iota(jnp.int32, (16,), 0)

        @pl.loop(0, nck, step=2)
        def _(jj):
            for jo in range(2):
                j = jj + jo
                jh = lax.shift_right_logical(j, 1)
                off = lax.rem(j, 2) * ck
                pltpu.make_async_copy(
                    in_hbm.at[colb.at[jh, pl.ds(off, ck)]],
                    rows[jo], gsems[jo]).wait()

                for kk in range(ck // 16):
                    r16 = rowb[jh, pl.ds(off + kk * 16, 16)]
                    rowcs[jo][pl.ds(kk * 16, 16)] = r16
                    rowds[jo][pl.ds(kk * 16, 16)] = (
                        lax.shift_right_logical(r16, 7))
                    excs[jo][pl.ds(kk * 16, 16)] = (
                        exb[jh, pl.ds(off + kk * 16, 16)])

                @pl.when(j > 0)
                def _():
                    pltpu.make_async_copy(vv, agg_sh.at[pl.ds(0, ck)],
                                          ssem).wait()
                    pltpu.make_async_copy(u0, agg_sh.at[pl.ds(0, ck)],
                                          ssem).wait()

                @pl.loop(0, ck)
                def _(i):
                    splat_i = jnp.full((16,), i, jnp.int32)
                    sv = plsc.load_gather(excs[jo], [splat_i])
                    rv = plsc.load_gather(rowcs[jo], [splat_i])
                    pv = lax.rem(rv, 128)
                    for kk in range(d // 16):
                        vv[i, pl.ds(kk * 16, 16)] = (
                            rows[jo][i, pl.ds(kk * 16, 16)] * sv)
                        u0[i, pl.ds(kk * 16, 16)] = jnp.where(
                            iota16 + (kk * 16) == pv, sv,
                            jnp.zeros((16,), jnp.float32))

                pltpu.async_copy(vv, agg_sh.at[rowcs[jo]], ssem, add=True)
                pltpu.async_copy(u0, den_sh.at[rowds[jo]], ssem, add=True)

                @pl.when(j + 2 < nck)
                def _():
                    jn = j + 2
                    jhn = lax.shift_right_logical(jn, 1)
                    offn = lax.rem(jn, 2) * ck
                    pltpu.async_copy(
                        in_hbm.at[colb.at[jhn, pl.ds(offn, ck)]],
                        rows[jo], gsems[jo])

        pltpu.make_async_copy(vv, agg_sh.at[pl.ds(0, ck)], ssem).wait()
        pltpu.make_async_copy(u0, agg_sh.at[pl.ds(0, ck)], ssem).wait()

        plsc.subcore_barrier()

        @pl.loop(0, iters_z)
        def _(z):
            c = sid + z * nsub

            @pl.when(c < nrchunk)
            def _():
                pltpu.sync_copy(agg_sh.at[pl.ds(c * rcp, rcp)],
                                agg_hbm.at[cid, pl.ds(c * rcp, rcp)])

        @pl.when(sid == 0)
        def _():
            pltpu.sync_copy(den_sh, den_hbm.at[cid])

    return k(ex2, row2, col2, input_)


def _finalize(input_, agg_pair, den_pair):
    n, d = input_.shape

    def body(x_ref, a_ref, dn_ref, o_ref):
        a = a_ref[0] + a_ref[1]
        den = dn_ref[0] + dn_ref[1]
        agg = jnp.where(den > 0.0, a / den, 0.0)
        out = x_ref[...] + agg
        o_ref[...] = jnp.where(out > 0.0, out, jnp.exp(out) - 1.0)

    return pl.pallas_call(
        body,
        grid=(n // _BN,),
        in_specs=[
            pl.BlockSpec((_BN, d), lambda i: (i, 0)),
            pl.BlockSpec((2, _BN, d), lambda i: (0, i, 0)),
            pl.BlockSpec((2, _BN, 1), lambda i: (0, i, 0)),
        ],
        out_specs=pl.BlockSpec((_BN, d), lambda i: (i, 0)),
        out_shape=jax.ShapeDtypeStruct((n, d), jnp.float32),
    )(input_, agg_pair, den_pair)


def kernel(input, triple, rel_table, W, conv_w, conv_b, bn1_gamma, bn1_beta,
           bn2_gamma, bn2_beta, fc_w):
    n, d = input.shape
    e = triple.shape[0]
    dc = d - 2
    eps = 1e-5

    row = triple[:, 0]
    rel = triple[:, 1]
    col = triple[:, 2]

    info = plsc.get_sparse_core_info()
    nw = info.num_cores * info.num_subcores
    per_w = nw * _CHUNK
    epad = -(-e // per_w) * per_w
    cpw = epad // per_w
    if cpw % 2:
        cpw += 1
        epad = cpw * per_w
    padlen = epad - e
    zpad = jnp.zeros((padlen,), jnp.int32)
    row2 = jnp.concatenate([row, zpad]).reshape(-1, _CHUNK)
    rel2 = jnp.concatenate([rel, zpad]).reshape(-1, _CHUNK)
    col2 = jnp.concatenate([col, zpad]).reshape(-1, _CHUNK)

    input_ = _matmul(input, W)

    Hb, Rb, Tb = _sc_gather(input_, rel_table, row2, rel2, col2)

    cwf = conv_w.reshape(4 * 9)
    stats = _stats_pass(Hb, Rb, Tb, cwf, e)[0]

    cnt1 = 3.0 * e * d
    mu1 = stats[0] / cnt1
    v1 = stats[1] / cnt1 - mu1 * mu1
    a1 = bn1_gamma[0] / jnp.sqrt(v1 + eps)
    cnt2 = float(e * dc)
    m0 = stats[2:10:2] / cnt2
    v0 = stats[3:10:2] / cnt2 - m0 * m0
    s_c = a1 * bn2_gamma / jnp.sqrt(a1 * a1 * v0 + eps)
    o_c = bn2_beta - s_c * m0

    fc2 = fc_w.reshape(4, dc)
    scal = jnp.zeros((16, 128), jnp.float32)
    scal = scal.at[0:4, :dc].set(jnp.broadcast_to(s_c[:, None], (4, dc)))
    scal = scal.at[4:8, :dc].set(jnp.broadcast_to(o_c[:, None], (4, dc)))
    scal = scal.at[8:12, :dc].set(fc2)

    ex = _score_pass(Hb, Rb, Tb, cwf, scal, e)[:, 0]
    ex2 = jnp.concatenate(
        [ex, jnp.zeros((padlen,), jnp.float32)]).reshape(-1, _CHUNK)

    agg_pair, den_out = _sc_aggregate(ex2, row2, col2, input_)
    den_pair = den_out.reshape(2, -1)[:, :n].reshape(2, n, 1)

    return _finalize(input_, agg_pair, den_pair)


# speedup vs baseline: 1.1618x; 1.1390x over previous
import dataclasses
import functools

import jax
import jax.numpy as jnp
from jax import lax
from jax.experimental import pallas as pl
from jax.experimental.pallas import tpu as pltpu
from jax.experimental.pallas import tpu_sc as plsc

_CHUNK = 128
_BE = 2000
_BN = 2000


def _matmul(x, w):
    n, d = x.shape

    def body(x_ref, w_ref, o_ref):
        o_ref[...] = jnp.dot(x_ref[...], w_ref[...],
                             preferred_element_type=jnp.float32)

    return pl.pallas_call(
        body,
        grid=(n // _BN,),
        in_specs=[
            pl.BlockSpec((_BN, d), lambda i: (i, 0)),
            pl.BlockSpec((d, d), lambda i: (0, 0)),
        ],
        out_specs=pl.BlockSpec((_BN, d), lambda i: (i, 0)),
        out_shape=jax.ShapeDtypeStruct((n, d), jnp.float32),
    )(x, w)


def _sc_gather(table_h, table_r, hidx, ridx, cidx):
    e = hidx.shape[0]
    nchunk = e // _CHUNK
    d = table_h.shape[1]
    info = plsc.get_sparse_core_info()
    nw = info.num_cores * info.num_subcores
    iters = nchunk // nw
    mesh = plsc.VectorSubcoreMesh(core_axis_name="c", subcore_axis_name="s")
    out_t = jax.ShapeDtypeStruct((e, d), jnp.float32)

    @functools.partial(
        pl.kernel, mesh=mesh,
        out_type=[out_t, out_t, out_t],
        scratch_types=[
            pltpu.VMEM((_CHUNK,), jnp.int32),
            pltpu.VMEM((_CHUNK,), jnp.int32),
            pltpu.VMEM((_CHUNK,), jnp.int32),
            pltpu.VMEM((_CHUNK, d), jnp.float32),
            pltpu.VMEM((_CHUNK, d), jnp.float32),
            pltpu.VMEM((_CHUNK, d), jnp.float32),
            pltpu.SemaphoreType.DMA,
            pltpu.SemaphoreType.DMA,
            pltpu.SemaphoreType.DMA,
        ],
    )
    def k(th_hbm, tr_hbm, hi_hbm, ri_hbm, ci_hbm, ho_hbm, ro_hbm, to_hbm,
          hi_v, ri_v, ci_v, hb, rb, tb, s0, s1, s2):
        w = lax.axis_index("s") * info.num_cores + lax.axis_index("c")

        @pl.loop(0, iters)
        def _(jj):
            j = w + jj * nw
            base = j * _CHUNK
            pltpu.sync_copy(hi_hbm.at[pl.ds(base, _CHUNK)], hi_v)
            pltpu.sync_copy(ri_hbm.at[pl.ds(base, _CHUNK)], ri_v)
            pltpu.sync_copy(ci_hbm.at[pl.ds(base, _CHUNK)], ci_v)
            c0 = pltpu.async_copy(th_hbm.at[hi_v], hb, s0)
            c1 = pltpu.async_copy(tr_hbm.at[ri_v], rb, s1)
            c2 = pltpu.async_copy(th_hbm.at[ci_v], tb, s2)
            c0.wait()
            c1.wait()
            c2.wait()
            pltpu.sync_copy(hb, ho_hbm.at[pl.ds(base, _CHUNK)])
            pltpu.sync_copy(rb, ro_hbm.at[pl.ds(base, _CHUNK)])
            pltpu.sync_copy(tb, to_hbm.at[pl.ds(base, _CHUNK)])

    return k(table_h, table_r, hidx, ridx, cidx)


_DL = 512


def _conv_l(h_ref, r_ref, t_ref, wh_ref, wr_ref, wt_ref):
    hb = h_ref[...].astype(jnp.bfloat16)
    rb = r_ref[...].astype(jnp.bfloat16)
    tb = t_ref[...].astype(jnp.bfloat16)
    return (jnp.dot(hb, wh_ref[...], preferred_element_type=jnp.float32)
            + jnp.dot(rb, wr_ref[...], preferred_element_type=jnp.float32)
            + jnp.dot(tb, wt_ref[...], preferred_element_type=jnp.float32))


def _stats_pass(H, R, T, wh, wr, wt, ne):
    d = H.shape[1]
    dc = d - 2
    nb = ne // _BE

    def body(h_ref, r_ref, t_ref, wh_ref, wr_ref, wt_ref, o_ref):
        i = pl.program_id(0)

        @pl.when(i == 0)
        def _():
            o_ref[...] = jnp.zeros_like(o_ref)

        h = h_ref[...]
        r = r_ref[...]
        t = t_ref[...]
        vals = [
            jnp.sum(h) + jnp.sum(r) + jnp.sum(t),
            jnp.sum(h * h) + jnp.sum(r * r) + jnp.sum(t * t),
        ]
        L = _conv_l(h_ref, r_ref, t_ref, wh_ref, wr_ref, wt_ref)
        cs = jnp.sum(L, axis=0, keepdims=True)
        cs2 = jnp.sum(L * L, axis=0, keepdims=True)
        ch = lax.broadcasted_iota(jnp.int32, (1, _DL), 1) // dc
        for c in range(4):
            vals.append(jnp.sum(jnp.where(ch == c, cs, 0.0)))
            vals.append(jnp.sum(jnp.where(ch == c, cs2, 0.0)))
        lane = lax.broadcasted_iota(jnp.int32, (1, 128), 1)
        p = jnp.zeros((1, 128), jnp.float32)
        for k, v in enumerate(vals):
            p = p + jnp.where(lane == k, v, 0.0)
        o_ref[...] += p

    return pl.pallas_call(
        body,
        grid=(nb,),
        in_specs=[
            pl.BlockSpec((_BE, d), lambda i: (i, 0)),
            pl.BlockSpec((_BE, d), lambda i: (i, 0)),
            pl.BlockSpec((_BE, d), lambda i: (i, 0)),
            pl.BlockSpec((d, _DL), lambda i: (0, 0)),
            pl.BlockSpec((d, _DL), lambda i: (0, 0)),
            pl.BlockSpec((d, _DL), lambda i: (0, 0)),
        ],
        out_specs=pl.BlockSpec((1, 128), lambda i: (0, 0)),
        out_shape=jax.ShapeDtypeStruct((1, 128), jnp.float32),
    )(H, R, T, wh, wr, wt)


def _score_pass(H, R, T, wh, wr, wt, scal, ne):
    d = H.shape[1]
    nb = ne // _BE

    def body(h_ref, r_ref, t_ref, wh_ref, wr_ref, wt_ref, scal_ref, o_ref):
        L = _conv_l(h_ref, r_ref, t_ref, wh_ref, wr_ref, wt_ref)
        z = L * scal_ref[0:1, :] + scal_ref[1:2, :]
        tot = jnp.maximum(z, 0.0) * scal_ref[2:3, :]
        ev = jnp.sum(tot, axis=1, keepdims=True)
        ev = jnp.where(ev >= 0.0, ev, 0.01 * ev)
        o_ref[...] = jnp.exp(ev)

    return pl.pallas_call(
        body,
        grid=(nb,),
        in_specs=[
            pl.BlockSpec((_BE, d), lambda i: (i, 0)),
            pl.BlockSpec((_BE, d), lambda i: (i, 0)),
            pl.BlockSpec((_BE, d), lambda i: (i, 0)),
            pl.BlockSpec((d, _DL), lambda i: (0, 0)),
            pl.BlockSpec((d, _DL), lambda i: (0, 0)),
            pl.BlockSpec((d, _DL), lambda i: (0, 0)),
            pl.BlockSpec((8, _DL), lambda i: (0, 0)),
        ],
        out_specs=pl.BlockSpec((_BE, 1), lambda i: (i, 0)),
        out_shape=jax.ShapeDtypeStruct((ne, 1), jnp.float32),
    )(H, R, T, wh, wr, wt, scal)


def _sc_aggregate(ex2, row2, col2, input_):
    nchunk, _ = ex2.shape
    ck = 64
    n, d = input_.shape
    info = plsc.get_sparse_core_info()
    ncores, nsub = info.num_cores, info.num_subcores
    nw = ncores * nsub
    cpw = nchunk // nw
    nck = cpw * 2
    rcp = 40
    nrchunk = n // rcp
    iters_z = (nrchunk + nsub - 1) // nsub
    ndr = -(-n // d) + (-(-n // d)) % 8
    mesh = plsc.VectorSubcoreMesh(core_axis_name="c", subcore_axis_name="s")
    cp = pltpu.CompilerParams()
    if "needs_layout_passes" in pltpu.CompilerParams.__dataclass_fields__:
        cp = dataclasses.replace(cp, needs_layout_passes=False)

    @functools.partial(
        pl.kernel, mesh=mesh, compiler_params=cp,
        out_type=[jax.ShapeDtypeStruct((ncores, n, d), jnp.float32),
                  jax.ShapeDtypeStruct((ncores, ndr, d), jnp.float32)],
        scratch_types=[
            pltpu.VMEM((cpw, 128), jnp.float32),
            pltpu.VMEM((cpw, 128), jnp.int32),
            pltpu.VMEM((cpw, 128), jnp.int32),
            pltpu.VMEM((ck,), jnp.float32),
            pltpu.VMEM((ck,), jnp.float32),
            pltpu.VMEM((ck,), jnp.int32),
            pltpu.VMEM((ck,), jnp.int32),
            pltpu.VMEM((ck,), jnp.int32),
            pltpu.VMEM((ck,), jnp.int32),
        ] + [pltpu.VMEM((ck, d), jnp.float32)] * 4 + [
            pltpu.VMEM_SHARED((n, d), jnp.float32),
            pltpu.VMEM_SHARED((ndr, d), jnp.float32),
            pltpu.SemaphoreType.DMA,
            pltpu.SemaphoreType.DMA,
            pltpu.SemaphoreType.DMA,
        ],
    )
    def k(ex_hbm, row_hbm, col_hbm, in_hbm, agg_hbm, den_hbm,
          exb, rowb, colb, ec0, ec1, rc0, rc1, rd0, rd1, r0, r1, vv, u0,
          agg_sh, den_sh, gs0, gs1, ssem):
        cid = lax.axis_index("c")
        sid = lax.axis_index("s")
        w = sid * ncores + cid
        c0 = w * cpw
        pltpu.sync_copy(ex_hbm.at[pl.ds(c0, cpw)], exb)
        pltpu.sync_copy(row_hbm.at[pl.ds(c0, cpw)], rowb)
        pltpu.sync_copy(col_hbm.at[pl.ds(c0, cpw)], colb)

        @pl.loop(0, ck)
        def _(i):
            for kk in range(d // 16):
                u0[i, pl.ds(kk * 16, 16)] = jnp.zeros((16,), jnp.float32)

        @pl.loop(0, iters_z)
        def _(z):
            c = sid + z * nsub

            @pl.when(c < nrchunk)
            def _():
                pltpu.sync_copy(u0.at[pl.ds(0, rcp)],
                                agg_sh.at[pl.ds(c * rcp, rcp)])

        @pl.when(sid == 0)
        def _():
            pltpu.sync_copy(u0.at[pl.ds(0, ndr - ck)],
                            den_sh.at[pl.ds(0, ndr - ck)])
            pltpu.sync_copy(u0.at[pl.ds(0, ck)], den_sh.at[pl.ds(ndr - ck, ck)])

        plsc.subcore_barrier()

        rows = (r0, r1)
        excs = (ec0, ec1)
        rowcs = (rc0, rc1)
        rowds = (rd0, rd1)
        gsems = (gs0, gs1)
        for jo in range(2):
            pltpu.async_copy(in_hbm.at[colb.at[0, pl.ds(jo * ck, ck)]],
                             rows[jo], gsems[jo])
        iota16 = lax.broadcasted_iota(jnp.int32, (16,), 0)

        @pl.loop(0, nck, step=2)
        def _(jj):
            for jo in range(2):
                j = jj + jo
                jh = lax.shift_right_logical(j, 1)
                off = lax.rem(j, 2) * ck
                pltpu.make_async_copy(
                    in_hbm.at[colb.at[jh, pl.ds(off, ck)]],
                    rows[jo], gsems[jo]).wait()

                for kk in range(ck // 16):
                    r16 = rowb[jh, pl.ds(off + kk * 16, 16)]
                    rowcs[jo][pl.ds(kk * 16, 16)] = r16
                    rowds[jo][pl.ds(kk * 16, 16)] = (
                        lax.shift_right_logical(r16, 7))
                    excs[jo][pl.ds(kk * 16, 16)] = (
                        exb[jh, pl.ds(off + kk * 16, 16)])

                @pl.when(j > 0)
                def _():
                    pltpu.make_async_copy(vv, agg_sh.at[pl.ds(0, ck)],
                                          ssem).wait()
                    pltpu.make_async_copy(u0, agg_sh.at[pl.ds(0, ck)],
                                          ssem).wait()

                @pl.loop(0, ck)
                def _(i):
                    splat_i = jnp.full((16,), i, jnp.int32)
                    sv = plsc.load_gather(excs[jo], [splat_i])
                    rv = plsc.load_gather(rowcs[jo], [splat_i])
                    pv = lax.rem(rv, 128)
                    for kk in range(d // 16):
                        vv[i, pl.ds(kk * 16, 16)] = (
                            rows[jo][i, pl.ds(kk * 16, 16)] * sv)
                        u0[i, pl.ds(kk * 16, 16)] = jnp.where(
                            iota16 + (kk * 16) == pv, sv,
                            jnp.zeros((16,), jnp.float32))

                pltpu.async_copy(vv, agg_sh.at[rowcs[jo]], ssem, add=True)
                pltpu.async_copy(u0, den_sh.at[rowds[jo]], ssem, add=True)

                @pl.when(j + 2 < nck)
                def _():
                    jn = j + 2
                    jhn = lax.shift_right_logical(jn, 1)
                    offn = lax.rem(jn, 2) * ck
                    pltpu.async_copy(
                        in_hbm.at[colb.at[jhn, pl.ds(offn, ck)]],
                        rows[jo], gsems[jo])

        pltpu.make_async_copy(vv, agg_sh.at[pl.ds(0, ck)], ssem).wait()
        pltpu.make_async_copy(u0, agg_sh.at[pl.ds(0, ck)], ssem).wait()

        plsc.subcore_barrier()

        @pl.loop(0, iters_z)
        def _(z):
            c = sid + z * nsub

            @pl.when(c < nrchunk)
            def _():
                pltpu.sync_copy(agg_sh.at[pl.ds(c * rcp, rcp)],
                                agg_hbm.at[cid, pl.ds(c * rcp, rcp)])

        @pl.when(sid == 0)
        def _():
            pltpu.sync_copy(den_sh, den_hbm.at[cid])

    return k(ex2, row2, col2, input_)


def _finalize(input_, agg_pair, den_pair):
    n, d = input_.shape

    def body(x_ref, a_ref, dn_ref, o_ref):
        a = a_ref[0] + a_ref[1]
        den = dn_ref[0] + dn_ref[1]
        agg = jnp.where(den > 0.0, a / den, 0.0)
        out = x_ref[...] + agg
        o_ref[...] = jnp.where(out > 0.0, out, jnp.exp(out) - 1.0)

    return pl.pallas_call(
        body,
        grid=(n // _BN,),
        in_specs=[
            pl.BlockSpec((_BN, d), lambda i: (i, 0)),
            pl.BlockSpec((2, _BN, d), lambda i: (0, i, 0)),
            pl.BlockSpec((2, _BN, 1), lambda i: (0, i, 0)),
        ],
        out_specs=pl.BlockSpec((_BN, d), lambda i: (i, 0)),
        out_shape=jax.ShapeDtypeStruct((n, d), jnp.float32),
    )(input_, agg_pair, den_pair)


def kernel(input, triple, rel_table, W, conv_w, conv_b, bn1_gamma, bn1_beta,
           bn2_gamma, bn2_beta, fc_w):
    n, d = input.shape
    e = triple.shape[0]
    dc = d - 2
    eps = 1e-5

    row = triple[:, 0]
    rel = triple[:, 1]
    col = triple[:, 2]

    info = plsc.get_sparse_core_info()
    nw = info.num_cores * info.num_subcores
    per_w = nw * _CHUNK
    epad = -(-e // per_w) * per_w
    cpw = epad // per_w
    if cpw % 2:
        cpw += 1
        epad = cpw * per_w
    padlen = epad - e
    zpad = jnp.zeros((padlen,), jnp.int32)
    rowf = jnp.concatenate([row, zpad])
    relf = jnp.concatenate([rel, zpad])
    colf = jnp.concatenate([col, zpad])
    row2 = rowf.reshape(-1, _CHUNK)
    col2 = colf.reshape(-1, _CHUNK)

    input_ = _matmul(input, W)

    Hb, Rb, Tb = _sc_gather(input_, rel_table, rowf, relf, colf)

    ar = jnp.arange(dc)
    wmats = []
    for kj in range(3):
        wm = jnp.zeros((d, _DL), jnp.float32)
        for c in range(4):
            for ki in range(3):
                wm = wm.at[ar + ki, c * dc + ar].set(conv_w[c, 0, ki, kj])
        wmats.append(wm.astype(jnp.bfloat16))
    wh, wr, wt = wmats

    stats = _stats_pass(Hb, Rb, Tb, wh, wr, wt, e)[0]

    cnt1 = 3.0 * e * d
    mu1 = stats[0] / cnt1
    v1 = stats[1] / cnt1 - mu1 * mu1
    a1 = bn1_gamma[0] / jnp.sqrt(v1 + eps)
    cnt2 = float(e * dc)
    m0 = stats[2:10:2] / cnt2
    v0 = stats[3:10:2] / cnt2 - m0 * m0
    s_c = a1 * bn2_gamma / jnp.sqrt(a1 * a1 * v0 + eps)
    o_c = bn2_beta - s_c * m0

    scal = jnp.zeros((8, _DL), jnp.float32)
    scal = scal.at[0, :4 * dc].set(jnp.repeat(s_c, dc))
    scal = scal.at[1, :4 * dc].set(jnp.repeat(o_c, dc))
    scal = scal.at[2, :4 * dc].set(fc_w)

    ex = _score_pass(Hb, Rb, Tb, wh, wr, wt, scal, e)[:, 0]
    ex2 = jnp.concatenate(
        [ex, jnp.zeros((padlen,), jnp.float32)]).reshape(-1, _CHUNK)

    agg_pair, den_out = _sc_aggregate(ex2, row2, col2, input_)
    den_pair = den_out.reshape(2, -1)[:, :n].reshape(2, n, 1)

    return _finalize(input_, agg_pair, den_pair)
